# baseline (speedup 1.0000x reference)
import functools

import jax
import jax.numpy as jnp
from jax import lax
from jax.experimental import pallas as pl
from jax.experimental.pallas import tpu as pltpu
from jax.experimental.pallas import tpu_sc as plsc

BATCH = 16384
HIST = 200
EMBED_DIM = 32
TOTAL = BATCH * HIST

NUM_CORES = 2
NUM_SUBCORES = 16
NUM_WORKERS = NUM_CORES * NUM_SUBCORES

HC = HIST // 8
BT = BATCH // 128
BT_PER_W = BT // NUM_WORKERS
BLOCK = 8 * 128
NBLOCKS = HC * BT_PER_W
NPAIRS = NBLOCKS // 2
OUT_WORDS = TOTAL * EMBED_DIM

_MESH = plsc.VectorSubcoreMesh(core_axis_name="c", subcore_axis_name="s")


@functools.partial(
    pl.kernel,
    mesh=_MESH,
    out_type=jax.ShapeDtypeStruct((OUT_WORDS,), jnp.float32),
    scratch_types=[
        pltpu.VMEM((BLOCK,), jnp.int32),
        pltpu.VMEM((BLOCK,), jnp.int32),
        pltpu.VMEM((BLOCK, EMBED_DIM), jnp.float32),
        pltpu.VMEM((BLOCK, EMBED_DIM), jnp.float32),
        pltpu.VMEM((BLOCK * EMBED_DIM,), jnp.float32),
        pltpu.SemaphoreType.DMA((2,)),
        pltpu.SemaphoreType.DMA((2,)),
        pltpu.SemaphoreType.DMA,
    ],
    compiler_params=pltpu.CompilerParams(use_tc_tiling_on_sc=False,
                                         needs_layout_passes=False),
)
def _sc_embed(table_hbm, info4_hbm, out_hbm, idx_a, idx_b, rows_a, rows_b,
              trans_v, idx_sem, gat_sem, st_sem):
    wid = lax.axis_index("s") * NUM_CORES + lax.axis_index("c")
    bt0 = wid * BT_PER_W

    iota16 = lax.iota(jnp.int32, 16)
    i128 = iota16 * 128
    idx_ref = (idx_a, idx_b)
    rows_ref = (rows_a, rows_b)

    def hcbt(g):
        hc = g // BT_PER_W
        return hc, bt0 + (g - hc * BT_PER_W)

    def idx_load(g, s):
        hc, bt = hcbt(g)
        pltpu.async_copy(info4_hbm.at[hc, bt], idx_ref[s], idx_sem.at[s])

    def idx_wait(g, s):
        hc, bt = hcbt(g)
        pltpu.make_async_copy(info4_hbm.at[hc, bt], idx_ref[s],
                              idx_sem.at[s]).wait()

    def gather(s):
        pltpu.async_copy(table_hbm.at[idx_ref[s]], rows_ref[s],
                         gat_sem.at[s])

    def gather_wait(s):
        pltpu.make_async_copy(table_hbm.at[idx_ref[s]], rows_ref[s],
                              gat_sem.at[s]).wait()

    def store_block(g):
        hc, bt = hcbt(g)
        for h8 in range(8):
            src_off = h8 * 4096
            dst_h = (hc * 8 + h8) * (4 * 128 * 1024)
            for et in range(4):
                pltpu.async_copy(
                    trans_v.at[pl.ds(src_off + et * 1024, 1024)],
                    out_hbm.at[pl.ds(dst_h + et * 131072 + bt * 1024, 1024)],
                    st_sem)

    def store_drain():
        for _ in range(32):
            pltpu.make_async_copy(
                trans_v.at[pl.ds(0, 1024)],
                out_hbm.at[pl.ds(0, 1024)],
                st_sem).wait()

    def transpose(s):
        rows = rows_ref[s]

        def tr_body(r, _):
            h8 = r // 128
            b = r - h8 * 128
            sb = h8 * 4096 + b
            ids0 = i128 + sb
            x0 = rows[r, pl.ds(0, 16)]
            x1 = rows[r, pl.ds(16, 16)]
            plsc.store_scatter(trans_v, [ids0], x0)
            plsc.store_scatter(trans_v, [ids0 + 2048], x1)
            return 0

        lax.fori_loop(0, BLOCK, tr_body, 0)

    idx_load(0, 0)
    idx_wait(0, 0)
    gather(0)
    idx_load(1, 1)

    def body(p, _):
        g0 = 2 * p
        g1 = g0 + 1
        not_last = p < NPAIRS - 1

        idx_wait(g1, 1)
        gather(1)

        gather_wait(0)

        @pl.when(not_last)
        def _():
            idx_load(g0 + 2, 0)

        @pl.when(p > 0)
        def _():
            store_drain()
        transpose(0)
        store_block(g0)

        gather_wait(1)

        @pl.when(not_last)
        def _():
            idx_load(g1 + 2, 1)

        @pl.when(not_last)
        def _():
            idx_wait(g0 + 2, 0)
            gather(0)

        store_drain()
        transpose(1)
        store_block(g1)
        return 0

    lax.fori_loop(0, NPAIRS, body, 0)

    store_drain()


def kernel(info, table):
    info4 = (info.T.reshape(HC, 8, BT, 128).transpose(0, 2, 1, 3)
             .reshape(HC, BT, BLOCK))
    out_flat = _sc_embed(table, info4)
    out5 = out_flat.reshape(HIST, 4, BT, 8, 128)
    return (out5.transpose(2, 4, 0, 1, 3)
            .reshape(BATCH, HIST, EMBED_DIM))


# speedup vs baseline: 1.1306x; 1.1306x over previous
import functools

import jax
import jax.numpy as jnp
from jax import lax
from jax.experimental import pallas as pl
from jax.experimental.pallas import tpu as pltpu
from jax.experimental.pallas import tpu_sc as plsc

BATCH = 16384
HIST = 200
EMBED_DIM = 32
TOTAL = BATCH * HIST

NUM_CORES = 2
NUM_SUBCORES = 16
NUM_WORKERS = NUM_CORES * NUM_SUBCORES

HC = HIST // 8
BT = BATCH // 128
BT_PER_W = BT // NUM_WORKERS
BLOCK = 8 * 128
NBLOCKS = HC * BT_PER_W
NPAIRS = NBLOCKS // 2
OUT_WORDS = TOTAL * EMBED_DIM

_MESH = plsc.VectorSubcoreMesh(core_axis_name="c", subcore_axis_name="s")


@functools.partial(
    pl.kernel,
    mesh=_MESH,
    out_type=jax.ShapeDtypeStruct((OUT_WORDS,), jnp.float32),
    scratch_types=[
        pltpu.VMEM((BLOCK,), jnp.int32),
        pltpu.VMEM((BLOCK,), jnp.int32),
        pltpu.VMEM((BLOCK, EMBED_DIM), jnp.float32),
        pltpu.VMEM((BLOCK, EMBED_DIM), jnp.float32),
        pltpu.VMEM((BLOCK * EMBED_DIM,), jnp.float32),
        pltpu.SemaphoreType.DMA((2,)),
        pltpu.SemaphoreType.DMA((2,)),
        pltpu.SemaphoreType.DMA,
    ],
    compiler_params=pltpu.CompilerParams(use_tc_tiling_on_sc=False,
                                         needs_layout_passes=False),
)
def _sc_embed(table_hbm, info4_hbm, out_hbm, idx_a, idx_b, rows_a, rows_b,
              trans_v, idx_sem, gat_sem, st_sem):
    wid = lax.axis_index("s") * NUM_CORES + lax.axis_index("c")
    bt0 = wid * BT_PER_W

    iota16 = lax.iota(jnp.int32, 16)
    i128 = iota16 * 128
    idx_ref = (idx_a, idx_b)
    rows_ref = (rows_a, rows_b)

    def hcbt(g):
        hc = g // BT_PER_W
        return hc, bt0 + (g - hc * BT_PER_W)

    def idx_load(g, s):
        hc, bt = hcbt(g)
        pltpu.async_copy(info4_hbm.at[hc, bt], idx_ref[s], idx_sem.at[s])

    def idx_wait(g, s):
        hc, bt = hcbt(g)
        pltpu.make_async_copy(info4_hbm.at[hc, bt], idx_ref[s],
                              idx_sem.at[s]).wait()

    def gather(s):
        pltpu.async_copy(table_hbm.at[idx_ref[s]], rows_ref[s],
                         gat_sem.at[s])

    def gather_wait(s):
        pltpu.make_async_copy(table_hbm.at[idx_ref[s]], rows_ref[s],
                              gat_sem.at[s]).wait()

    def store_block(g):
        hc, bt = hcbt(g)
        for h8 in range(8):
            src_off = h8 * 4096
            dst_h = (hc * 8 + h8) * (4 * 128 * 1024)
            for et in range(4):
                pltpu.async_copy(
                    trans_v.at[pl.ds(src_off + et * 1024, 1024)],
                    out_hbm.at[pl.ds(dst_h + et * 131072 + bt * 1024, 1024)],
                    st_sem)

    def store_drain():
        for _ in range(32):
            pltpu.make_async_copy(
                trans_v.at[pl.ds(0, 1024)],
                out_hbm.at[pl.ds(0, 1024)],
                st_sem).wait()

    def transpose(s):
        rows = rows_ref[s]

        for h8 in range(8):
            @plsc.parallel_loop(0, 128, unroll=8)
            def _tr(b, h8=h8):
                sb = h8 * 4096 + b
                ids0 = i128 + sb
                x0 = rows[h8 * 128 + b, pl.ds(0, 16)]
                x1 = rows[h8 * 128 + b, pl.ds(16, 16)]
                plsc.store_scatter(trans_v, [ids0], x0)
                plsc.store_scatter(trans_v, [ids0 + 2048], x1)

    idx_load(0, 0)
    idx_wait(0, 0)
    gather(0)
    idx_load(1, 1)

    def body(p, _):
        g0 = 2 * p
        g1 = g0 + 1
        not_last = p < NPAIRS - 1

        idx_wait(g1, 1)
        gather(1)

        gather_wait(0)

        @pl.when(not_last)
        def _():
            idx_load(g0 + 2, 0)

        @pl.when(p > 0)
        def _():
            store_drain()
        transpose(0)
        store_block(g0)

        gather_wait(1)

        @pl.when(not_last)
        def _():
            idx_load(g1 + 2, 1)

        @pl.when(not_last)
        def _():
            idx_wait(g0 + 2, 0)
            gather(0)

        store_drain()
        transpose(1)
        store_block(g1)
        return 0

    lax.fori_loop(0, NPAIRS, body, 0)

    store_drain()


def kernel(info, table):
    info4 = (info.T.reshape(HC, 8, BT, 128).transpose(0, 2, 1, 3)
             .reshape(HC, BT, BLOCK))
    out_flat = _sc_embed(table, info4)
    out5 = out_flat.reshape(HIST, 4, BT, 8, 128)
    return (out5.transpose(2, 4, 0, 1, 3)
            .reshape(BATCH, HIST, EMBED_DIM))


# speedup vs baseline: 1.8967x; 1.6777x over previous
import functools

import jax
import jax.numpy as jnp
from jax import lax
from jax.experimental import pallas as pl
from jax.experimental.pallas import tpu as pltpu
from jax.experimental.pallas import tpu_sc as plsc

BATCH = 16384
HIST = 200
EMBED_DIM = 32
TOTAL = BATCH * HIST

NUM_CORES = 2
NUM_SUBCORES = 16
NUM_WORKERS = NUM_CORES * NUM_SUBCORES

HC = HIST // 8
BT = BATCH // 128
BT_PER_W = BT // NUM_WORKERS
BLOCK = 8 * 128
NBLOCKS = HC * BT_PER_W
NPAIRS = NBLOCKS // 2
OUT_WORDS = TOTAL * EMBED_DIM

MPITCH = 136
MID_H8 = MPITCH * EMBED_DIM
NRING = 8

_MESH = plsc.VectorSubcoreMesh(core_axis_name="c", subcore_axis_name="s")


@functools.partial(
    pl.kernel,
    mesh=_MESH,
    out_type=jax.ShapeDtypeStruct((OUT_WORDS,), jnp.float32),
    scratch_types=[
        pltpu.VMEM((BLOCK,), jnp.int32),
        pltpu.VMEM((BLOCK,), jnp.int32),
        pltpu.VMEM((BLOCK, EMBED_DIM), jnp.float32),
        pltpu.VMEM((BLOCK, EMBED_DIM), jnp.float32),
        pltpu.VMEM((8 * MID_H8,), jnp.float32),
        pltpu.VMEM((NRING, 1024), jnp.float32),
        pltpu.SemaphoreType.DMA((2,)),
        pltpu.SemaphoreType.DMA((2,)),
        pltpu.SemaphoreType.DMA,
    ],
    compiler_params=pltpu.CompilerParams(use_tc_tiling_on_sc=False,
                                         needs_layout_passes=False),
)
def _sc_embed(table_hbm, info4_hbm, out_hbm, idx_a, idx_b, rows_a, rows_b,
              mid_v, ring_v, idx_sem, gat_sem, st_sem):
    wid = lax.axis_index("s") * NUM_CORES + lax.axis_index("c")
    bt0 = wid * BT_PER_W

    iota16 = lax.iota(jnp.int32, 16)
    i136 = iota16 * MPITCH
    idx_ref = (idx_a, idx_b)
    rows_ref = (rows_a, rows_b)

    def hcbt(g):
        hc = g // BT_PER_W
        return hc, bt0 + (g - hc * BT_PER_W)

    def idx_load(g, s):
        hc, bt = hcbt(g)
        pltpu.async_copy(info4_hbm.at[hc, bt], idx_ref[s], idx_sem.at[s])

    def idx_wait(g, s):
        hc, bt = hcbt(g)
        pltpu.make_async_copy(info4_hbm.at[hc, bt], idx_ref[s],
                              idx_sem.at[s]).wait()

    def gather(s):
        pltpu.async_copy(table_hbm.at[idx_ref[s]], rows_ref[s],
                         gat_sem.at[s])

    def gather_wait(s):
        pltpu.make_async_copy(table_hbm.at[idx_ref[s]], rows_ref[s],
                              gat_sem.at[s]).wait()

    def st_drain():
        pltpu.make_async_copy(ring_v.at[0], out_hbm.at[pl.ds(0, 1024)],
                              st_sem).wait()

    def pass1(s):
        rows = rows_ref[s]
        for h8 in range(8):
            @plsc.parallel_loop(0, 128, unroll=8)
            def _p1(b, h8=h8):
                sb = h8 * MID_H8 + b
                ids0 = i136 + sb
                x0 = rows[h8 * 128 + b, pl.ds(0, 16)]
                x1 = rows[h8 * 128 + b, pl.ds(16, 16)]
                plsc.store_scatter(mid_v, [ids0], x0)
                plsc.store_scatter(mid_v, [ids0 + 16 * MPITCH], x1)

    def pass2(g, first):
        hc, bt = hcbt(g)

        def run_body(r, _):
            h8 = r // 4
            et = r - h8 * 4
            slot = lax.rem(r, NRING)
            if first is None:
                st_drain()
            else:
                @pl.when(jnp.logical_or(jnp.logical_not(first),
                                        r >= NRING))
                def _():
                    st_drain()
            msrc = h8 * MID_H8 + et * (8 * MPITCH)
            for e8 in range(8):
                for b1 in range(8):
                    x = mid_v[pl.ds(msrc + e8 * MPITCH + b1 * 16, 16)]
                    ring_v[slot, pl.ds(e8 * 128 + b1 * 16, 16)] = x
            dst = ((hc * 8 + h8) * (4 * BT) + et * BT + bt) * 1024
            pltpu.async_copy(ring_v.at[slot],
                             out_hbm.at[pl.ds(dst, 1024)], st_sem)
            return 0

        lax.fori_loop(0, 32, run_body, 0)

    idx_load(0, 0)
    idx_wait(0, 0)
    gather(0)
    idx_load(1, 1)

    def body(p, _):
        g0 = 2 * p
        g1 = g0 + 1
        not_last = p < NPAIRS - 1
        first = p == 0

        idx_wait(g1, 1)
        gather(1)

        gather_wait(0)

        @pl.when(not_last)
        def _():
            idx_load(g0 + 2, 0)

        pass1(0)
        pass2(g0, first)

        gather_wait(1)

        @pl.when(not_last)
        def _():
            idx_load(g1 + 2, 1)

        @pl.when(not_last)
        def _():
            idx_wait(g0 + 2, 0)
            gather(0)

        pass1(1)
        pass2(g1, None)
        return 0

    lax.fori_loop(0, NPAIRS, body, 0)

    for _ in range(NRING):
        st_drain()


def kernel(info, table):
    info4 = (info.T.reshape(HC, 8, BT, 128).transpose(0, 2, 1, 3)
             .reshape(HC, BT, BLOCK))
    out_flat = _sc_embed(table, info4)
    out5 = out_flat.reshape(HIST, 4, BT, 8, 128)
    return (out5.transpose(2, 4, 0, 1, 3)
            .reshape(BATCH, HIST, EMBED_DIM))


# speedup vs baseline: 3.1840x; 1.6787x over previous
import functools

import jax
import jax.numpy as jnp
from jax import lax
from jax.experimental import pallas as pl
from jax.experimental.pallas import tpu as pltpu
from jax.experimental.pallas import tpu_sc as plsc

BATCH = 16384
HIST = 200
EMBED_DIM = 32
TOTAL = BATCH * HIST

NUM_CORES = 2
NUM_SUBCORES = 16
NUM_WORKERS = NUM_CORES * NUM_SUBCORES

HC = HIST // 8
BT = BATCH // 128
BT_PER_W = BT // NUM_WORKERS
BLOCK = 8 * 128
NBLOCKS = HC * BT_PER_W
NPAIRS = NBLOCKS // 2
OUT_WORDS = TOTAL * EMBED_DIM

MPITCH = 136
MID_H8 = MPITCH * EMBED_DIM
NRING = 8

_MESH = plsc.VectorSubcoreMesh(core_axis_name="c", subcore_axis_name="s")


@functools.partial(
    pl.kernel,
    mesh=_MESH,
    out_type=jax.ShapeDtypeStruct((HIST, 4, BT, 8, 128), jnp.float32),
    scratch_types=[
        pltpu.VMEM((BLOCK,), jnp.int32),
        pltpu.VMEM((BLOCK,), jnp.int32),
        pltpu.VMEM((BLOCK, EMBED_DIM), jnp.float32),
        pltpu.VMEM((BLOCK, EMBED_DIM), jnp.float32),
        pltpu.VMEM((8, EMBED_DIM, MPITCH), jnp.float32),
        pltpu.SemaphoreType.DMA((2,)),
        pltpu.SemaphoreType.DMA((2,)),
        pltpu.SemaphoreType.DMA,
    ],
    compiler_params=pltpu.CompilerParams(use_tc_tiling_on_sc=False,
                                         needs_layout_passes=False),
)
def _sc_embed(table_hbm, info4_hbm, out_hbm, idx_a, idx_b, rows_a, rows_b,
              mid_v, idx_sem, gat_sem, st_sem):
    wid = lax.axis_index("s") * NUM_CORES + lax.axis_index("c")
    bt0 = wid * BT_PER_W

    iota16 = lax.iota(jnp.int32, 16)
    idx_ref = (idx_a, idx_b)
    rows_ref = (rows_a, rows_b)

    def hcbt(g):
        hc = g // BT_PER_W
        return hc, bt0 + (g - hc * BT_PER_W)

    def idx_load(g, s):
        hc, bt = hcbt(g)
        pltpu.async_copy(info4_hbm.at[hc, bt], idx_ref[s], idx_sem.at[s])

    def idx_wait(g, s):
        hc, bt = hcbt(g)
        pltpu.make_async_copy(info4_hbm.at[hc, bt], idx_ref[s],
                              idx_sem.at[s]).wait()

    def gather(s):
        pltpu.async_copy(table_hbm.at[idx_ref[s]], rows_ref[s],
                         gat_sem.at[s])

    def gather_wait(s):
        pltpu.make_async_copy(table_hbm.at[idx_ref[s]], rows_ref[s],
                              gat_sem.at[s]).wait()

    def st_drain():
        pltpu.make_async_copy(
            mid_v.at[0, pl.ds(0, 8), pl.ds(0, 128)],
            out_hbm.at[0, 0, 0],
            st_sem).wait()

    def pass1(s):
        rows = rows_ref[s]
        for h8 in range(8):
            h8v = jnp.full((16,), h8, jnp.int32)

            @plsc.parallel_loop(0, 128, unroll=8)
            def _p1(b, h8=h8, h8v=h8v):
                bv = jnp.full((16,), b, jnp.int32)
                x0 = rows[h8 * 128 + b, pl.ds(0, 16)]
                x1 = rows[h8 * 128 + b, pl.ds(16, 16)]
                plsc.store_scatter(mid_v, [h8v, iota16, bv], x0)
                plsc.store_scatter(mid_v, [h8v, iota16 + 16, bv], x1)

    def pass2(g):
        hc, bt = hcbt(g)
        for h8 in range(8):
            for et in range(4):
                pltpu.async_copy(
                    mid_v.at[h8, pl.ds(et * 8, 8), pl.ds(0, 128)],
                    out_hbm.at[hc * 8 + h8, et, bt],
                    st_sem)

    def st_drain_block(first):
        if first is None:
            for _ in range(32):
                st_drain()
        else:
            @pl.when(jnp.logical_not(first))
            def _():
                for _ in range(32):
                    st_drain()

    idx_load(0, 0)
    idx_wait(0, 0)
    gather(0)
    idx_load(1, 1)

    def body(p, _):
        g0 = 2 * p
        g1 = g0 + 1
        not_last = p < NPAIRS - 1
        first = p == 0

        idx_wait(g1, 1)
        gather(1)

        gather_wait(0)

        @pl.when(not_last)
        def _():
            idx_load(g0 + 2, 0)

        st_drain_block(first)
        pass1(0)
        pass2(g0)

        gather_wait(1)

        @pl.when(not_last)
        def _():
            idx_load(g1 + 2, 1)

        @pl.when(not_last)
        def _():
            idx_wait(g0 + 2, 0)
            gather(0)

        st_drain_block(None)
        pass1(1)
        pass2(g1)
        return 0

    lax.fori_loop(0, NPAIRS, body, 0)

    st_drain_block(None)


def kernel(info, table):
    info4 = (info.T.reshape(HC, 8, BT, 128).transpose(0, 2, 1, 3)
             .reshape(HC, BT, BLOCK))
    out5 = _sc_embed(table, info4)
    return (out5.transpose(2, 4, 0, 1, 3)
            .reshape(BATCH, HIST, EMBED_DIM))
